# baseline (speedup 1.0000x reference)
import functools

import jax
import jax.numpy as jnp
from jax import lax
from jax.experimental import pallas as pl
from jax.experimental.pallas import tpu as pltpu
from jax.experimental.pallas import tpu_sc as plsc
from jax._src.pallas import mpmd

EMBED = 128
SEQ = 512
PE_LEN = 1024
TEC_I = 12
TEC_I_TOTAL = 32 * TEC_I
SCS_I = (SEQ - TEC_I_TOTAL) // 2
WIN = 528
WAVE = 16


@functools.partial(jax.jit, static_argnums=(1,))
def _run(pe, batch):
    info = plsc.get_sparse_core_info()
    nc = info.num_cores
    scalar_mesh = plsc.ScalarSubcoreMesh(axis_name="c", num_cores=nc)
    vector_mesh = plsc.VectorSubcoreMesh(core_axis_name="c",
                                         subcore_axis_name="s")

    def scs_fn(pe_hbm, out_hbm, pe_sh):
        c = lax.axis_index("c")
        i_base = TEC_I_TOTAL + SCS_I * c

        def inner(sem):
            pltpu.sync_copy(pe_hbm, pe_sh)
            tasks = SCS_I * batch
            prev = []
            for wave0 in range(0, tasks, WAVE):
                cur = []
                for t in range(wave0, min(wave0 + WAVE, tasks)):
                    i = i_base + t // batch
                    b = t % batch
                    row = (b * SEQ + i) * SEQ
                    cur.append(pltpu.async_copy(
                        pe_sh.at[pl.ds(SEQ - i, SEQ)],
                        out_hbm.at[pl.ds(row, SEQ)], sem))
                for h in prev:
                    h.wait()
                prev = cur
            for h in prev:
                h.wait()

        pl.run_scoped(inner, pltpu.SemaphoreType.DMA)

    def tec_fn(pe_hbm, out_hbm, pe_sh):
        del pe_sh
        t = lax.axis_index("s") * nc + lax.axis_index("c")
        i0 = t * TEC_I
        lo = ((SEQ - i0 - (TEC_I - 1)) // 8) * 8

        def inner(buf, sem):
            pltpu.sync_copy(pe_hbm.at[pl.ds(lo, WIN)], buf)
            handles = []
            for k in range(TEC_I):
                off = (SEQ - (i0 + k)) - lo
                src = buf.at[pl.ds(off, SEQ)]
                for b in range(batch):
                    row = (b * SEQ + i0 + k) * SEQ
                    handles.append(pltpu.async_copy(
                        src, out_hbm.at[pl.ds(row, SEQ)], sem))
            for h in handles:
                h.wait()

        pl.run_scoped(inner, pltpu.VMEM((WIN, EMBED), jnp.float32),
                      pltpu.SemaphoreType.DMA)

    call = mpmd.mpmd_map(
        [(vector_mesh, tec_fn), (scalar_mesh, scs_fn)],
        out_types=jax.ShapeDtypeStruct((batch * SEQ * SEQ, EMBED),
                                       jnp.float32),
        scratch_types=[pltpu.VMEM_SHARED((PE_LEN, EMBED), jnp.float32)],
    )
    return call(pe).reshape(batch, SEQ, SEQ, EMBED)


def kernel(x, pe):
    return _run(pe, x.shape[0])
